# baseline (speedup 1.0000x reference)
import functools

import jax
import jax.numpy as jnp
from jax import lax
from jax.experimental import pallas as pl
from jax.experimental.pallas import tpu as pltpu
from jax.experimental.pallas import tpu_sc as plsc

PROMPT_LENGTH = 64
EMBED_SIZE = 2048
BATCH = 4
SEQ_LEN = 2048

NW = 32
PRE_PER_W = (BATCH * PROMPT_LENGTH) // NW
BULK_PER_W = (BATCH * SEQ_LEN) // NW
CHUNK = 16
NBUF = 3
NCHUNKS = BULK_PER_W // CHUNK


def _sc_body(tokens_hbm, prompt_hbm, x_hbm, out_hbm,
             idx_v, pre_v, buf0, buf1, buf2, sem_pre, sem_in, sem_out):
    cid = lax.axis_index("c")
    sid = lax.axis_index("s")
    w = sid * 2 + cid

    r0 = w * BULK_PER_W
    b_bulk = r0 // SEQ_LEN
    row0 = r0 % SEQ_LEN
    bufs = (buf0, buf1, buf2)

    def in_copy(i):
        return pltpu.make_async_copy(
            x_hbm.at[b_bulk, pl.ds(row0 + i * CHUNK, CHUNK)],
            bufs[i % NBUF], sem_in.at[i % NBUF])

    def out_copy(i):
        return pltpu.make_async_copy(
            bufs[i % NBUF],
            out_hbm.at[b_bulk, pl.ds(PROMPT_LENGTH + row0 + i * CHUNK, CHUNK)],
            sem_out.at[i % NBUF])

    for i in range(NBUF):
        in_copy(i).start()

    p0 = w * PRE_PER_W
    b_pre = p0 // PROMPT_LENGTH
    s_pre = p0 % PROMPT_LENGTH
    pltpu.sync_copy(tokens_hbm.at[pl.ds(s_pre, PRE_PER_W)], idx_v)
    gat = pltpu.make_async_copy(prompt_hbm.at[idx_v], pre_v, sem_pre)
    gat.start()
    gat.wait()
    pltpu.sync_copy(pre_v, out_hbm.at[b_pre, pl.ds(s_pre, PRE_PER_W)])

    for i in range(NCHUNKS):
        in_copy(i).wait()
        out_copy(i).start()
        if i + NBUF < NCHUNKS:
            out_copy(i).wait()
            in_copy(i + NBUF).start()
    for i in range(max(0, NCHUNKS - NBUF), NCHUNKS):
        out_copy(i).wait()


def _make_sc_kernel():
    mesh = plsc.VectorSubcoreMesh(core_axis_name="c", subcore_axis_name="s",
                                  num_cores=2, num_subcores=16)
    return functools.partial(
        pl.kernel,
        mesh=mesh,
        out_type=jax.ShapeDtypeStruct(
            (BATCH, PROMPT_LENGTH + SEQ_LEN, EMBED_SIZE), jnp.float32),
        scratch_types=[
            pltpu.VMEM((PRE_PER_W,), jnp.int32),
            pltpu.VMEM((PRE_PER_W, EMBED_SIZE), jnp.float32),
            pltpu.VMEM((CHUNK, EMBED_SIZE), jnp.float32),
            pltpu.VMEM((CHUNK, EMBED_SIZE), jnp.float32),
            pltpu.VMEM((CHUNK, EMBED_SIZE), jnp.float32),
            pltpu.SemaphoreType.DMA,
            pltpu.SemaphoreType.DMA((NBUF,)),
            pltpu.SemaphoreType.DMA((NBUF,)),
        ],
    )(_sc_body)


def kernel(embedded_input, prompt_embedding, prompt_tokens):
    return _make_sc_kernel()(prompt_tokens, prompt_embedding, embedded_input)


# speedup vs baseline: 1.0733x; 1.0733x over previous
import functools

import jax
import jax.numpy as jnp
from jax import lax
from jax.experimental import pallas as pl
from jax.experimental.pallas import tpu as pltpu
from jax.experimental.pallas import tpu_sc as plsc

PROMPT_LENGTH = 64
EMBED_SIZE = 2048
BATCH = 4
SEQ_LEN = 2048

NW = 32
PRE_PER_W = (BATCH * PROMPT_LENGTH) // NW

CHUNK = 1024
NBUF = 6


def _sc_prefix_body(tokens_hbm, prompt_hbm, out_hbm, idx_v, pre_v, sem_pre):
    cid = lax.axis_index("c")
    sid = lax.axis_index("s")
    w = sid * 2 + cid
    p0 = w * PRE_PER_W
    b_pre = p0 // PROMPT_LENGTH
    s_pre = p0 % PROMPT_LENGTH
    pltpu.sync_copy(tokens_hbm.at[pl.ds(s_pre, PRE_PER_W)], idx_v)
    gat = pltpu.make_async_copy(prompt_hbm.at[idx_v], pre_v, sem_pre)
    gat.start()
    gat.wait()
    pltpu.sync_copy(pre_v, out_hbm.at[b_pre, pl.ds(s_pre, PRE_PER_W)])


def _sc_prefix(prompt_tokens, prompt_embedding):
    mesh = plsc.VectorSubcoreMesh(core_axis_name="c", subcore_axis_name="s",
                                  num_cores=2, num_subcores=16)
    k = functools.partial(
        pl.kernel,
        mesh=mesh,
        out_type=jax.ShapeDtypeStruct(
            (BATCH, PROMPT_LENGTH + SEQ_LEN, EMBED_SIZE), jnp.float32),
        scratch_types=[
            pltpu.VMEM((PRE_PER_W,), jnp.int32),
            pltpu.VMEM((PRE_PER_W, EMBED_SIZE), jnp.float32),
            pltpu.SemaphoreType.DMA,
        ],
    )(_sc_prefix_body)
    return k(prompt_tokens, prompt_embedding)


def _tc_bulk_body(x_hbm, out_in_hbm, out_hbm, bufs_vmem, sem_in, sem_out):
    del out_in_hbm
    batch = x_hbm.shape[0]
    seq_len = x_hbm.shape[1]
    chunks_per_batch = seq_len // CHUNK
    n_chunks = batch * chunks_per_batch

    def in_copy(i):
        b, c = divmod(i, chunks_per_batch)
        return pltpu.make_async_copy(
            x_hbm.at[b, pl.ds(c * CHUNK, CHUNK)],
            bufs_vmem.at[i % NBUF],
            sem_in.at[i % NBUF])

    def out_copy(i):
        b, c = divmod(i, chunks_per_batch)
        return pltpu.make_async_copy(
            bufs_vmem.at[i % NBUF],
            out_hbm.at[b, pl.ds(PROMPT_LENGTH + c * CHUNK, CHUNK)],
            sem_out.at[i % NBUF])

    for i in range(min(NBUF, n_chunks)):
        in_copy(i).start()
    for i in range(n_chunks):
        in_copy(i).wait()
        out_copy(i).start()
        if i + NBUF < n_chunks:
            out_copy(i).wait()
            in_copy(i + NBUF).start()
    for i in range(max(0, n_chunks - NBUF), n_chunks):
        out_copy(i).wait()


def _tc_bulk(embedded_input, out_prev):
    batch, seq_len, emb = embedded_input.shape
    return pl.pallas_call(
        _tc_bulk_body,
        in_specs=[
            pl.BlockSpec(memory_space=pltpu.MemorySpace.HBM),
            pl.BlockSpec(memory_space=pltpu.MemorySpace.HBM),
        ],
        out_specs=pl.BlockSpec(memory_space=pltpu.MemorySpace.HBM),
        out_shape=jax.ShapeDtypeStruct(
            (batch, PROMPT_LENGTH + seq_len, emb), jnp.float32),
        input_output_aliases={1: 0},
        scratch_shapes=[
            pltpu.VMEM((NBUF, CHUNK, EMBED_SIZE), jnp.float32),
            pltpu.SemaphoreType.DMA((NBUF,)),
            pltpu.SemaphoreType.DMA((NBUF,)),
        ],
    )(embedded_input, out_prev)


def kernel(embedded_input, prompt_embedding, prompt_tokens):
    out_prefix = _sc_prefix(prompt_tokens, prompt_embedding)
    return _tc_bulk(embedded_input, out_prefix)
